# Initial kernel scaffold; baseline (speedup 1.0000x reference)
import jax
import jax.numpy as jnp
from jax.experimental import pallas as pl


def kernel(query_X, key_X, value_X, Wq, bq, Wk, bk, Wv, bv):
    raise NotImplementedError("write your pallas kernel here")


# speedup vs baseline: 48.5159x; 48.5159x over previous
import functools

import jax
import jax.numpy as jnp
import numpy as np
from jax.experimental import pallas as pl


@functools.lru_cache(maxsize=None)
def _mask_bias(board):
    N = int(np.prod(board))
    coords = np.stack(np.unravel_index(np.arange(N), board), axis=-1)
    delta = np.abs(coords[None, :, :] - coords[:, None, :])
    mx = delta.max(axis=-1)
    connected = (mx > 0) & np.all((delta == 0) | (delta == mx[..., None]), axis=-1)
    return np.where(connected, 0.0, -1e30).astype(np.float32)


def _attn_kernel(xq_ref, xk_ref, xv_ref, wq_ref, bq_ref, wk_ref, bk_ref,
                 wv_ref, bv_ref, bias_ref, out_ref, *, scale):
    xq = xq_ref[0]
    xk = xk_ref[0]
    xv = xv_ref[0]
    q = jnp.dot(xq, wq_ref[...], preferred_element_type=jnp.float32) + bq_ref[...]
    k = jnp.dot(xk, wk_ref[...], preferred_element_type=jnp.float32) + bk_ref[...]
    v = jnp.dot(xv, wv_ref[...], preferred_element_type=jnp.float32) + bv_ref[...]
    s = jax.lax.dot_general(q, k, (((1,), (1,)), ((), ())),
                            preferred_element_type=jnp.float32)
    s = s * scale + bias_ref[...]
    m = jnp.max(s, axis=1, keepdims=True)
    p = jnp.exp(s - m)
    denom = jnp.sum(p, axis=1, keepdims=True)
    o = jnp.dot(p, v, preferred_element_type=jnp.float32)
    out_ref[0] = o / denom


def kernel(query_X, key_X, value_X, Wq, bq, Wk, bk, Wv, bv):
    B = query_X.shape[0]
    board = tuple(int(d) for d in query_X.shape[1:-1])
    in_dim = query_X.shape[-1]
    cmp_dim = Wq.shape[1]
    out_dim = Wv.shape[1]
    N = int(np.prod(board))

    bias = jnp.asarray(_mask_bias(board))
    xq = query_X.reshape(B, N, in_dim)
    xk = key_X.reshape(B, N, in_dim)
    xv = value_X.reshape(B, N, in_dim)

    batch_spec = pl.BlockSpec((1, N, in_dim), lambda b: (b, 0, 0))
    full = lambda shape: pl.BlockSpec(shape, lambda b: (0,) * len(shape))

    out = pl.pallas_call(
        functools.partial(_attn_kernel, scale=1.0 / (cmp_dim ** 0.5)),
        grid=(B,),
        in_specs=[
            batch_spec, batch_spec, batch_spec,
            full((in_dim, cmp_dim)), full((1, cmp_dim)),
            full((in_dim, cmp_dim)), full((1, cmp_dim)),
            full((in_dim, out_dim)), full((1, out_dim)),
            full((N, N)),
        ],
        out_specs=pl.BlockSpec((1, N, out_dim), lambda b: (b, 0, 0)),
        out_shape=jax.ShapeDtypeStruct((B, N, out_dim), jnp.float32),
    )(xq, xk, xv, Wq, bq.reshape(1, cmp_dim), Wk, bk.reshape(1, cmp_dim),
      Wv, bv.reshape(1, out_dim), bias)

    return out.reshape((B,) + board + (out_dim,))
